# baseline (speedup 1.0000x reference)
import functools

import jax
import jax.numpy as jnp
import numpy as np
from jax import lax
from jax.experimental import pallas as pl
from jax.experimental.pallas import tpu as pltpu

_K = 3
_TINY = np.float32(np.finfo(np.float32).tiny)
_ROT_A = (13, 15, 26, 6)
_ROT_B = (17, 29, 16, 24)


def _threefry_rounds(x0, x1, rots):
    for r in rots:
        x0 = x0 + x1
        x1 = lax.shift_left(x1, np.int32(r)) | lax.shift_right_logical(
            x1, np.int32(32 - r)
        )
        x1 = x1 ^ x0
    return x0, x1


def _gumbel_from_index(idx):
    ks1 = np.int32(42)
    ks2 = np.int32(0x1BD11BDA ^ 42)
    x1 = idx + ks1
    x0 = x1
    x1 = (
        lax.shift_left(x1, np.int32(13))
        | lax.shift_right_logical(x1, np.int32(19))
    ) ^ x0
    x0, x1 = _threefry_rounds(x0, x1, _ROT_A[1:])
    x0, x1 = x0 + ks1, x1 + np.int32(ks2 + 1)
    x0, x1 = _threefry_rounds(x0, x1, _ROT_B)
    x0, x1 = x0 + ks2, x1 + np.int32(2)
    x0, x1 = _threefry_rounds(x0, x1, _ROT_A)
    x0, x1 = x0, x1 + np.int32(ks1 + 3)
    x0, x1 = _threefry_rounds(x0, x1, _ROT_B)
    x0, x1 = x0 + ks1, x1 + np.int32(ks2 + 4)
    x0, x1 = _threefry_rounds(x0, x1, _ROT_A)
    x0, x1 = x0 + ks2, x1 + np.int32(5)
    bits = x0 ^ x1
    float_bits = lax.shift_right_logical(bits, np.int32(9)) | np.int32(0x3F800000)
    f = lax.bitcast_convert_type(float_bits, jnp.float32) - np.float32(1.0)
    u = jnp.maximum(_TINY, f * (np.float32(1.0) - _TINY) + _TINY)
    return -jnp.log(-jnp.log(u))


def _pool_kernel(x_ref, o_ref, *, oh, ow, w, pack):
    g = pl.program_id(0)
    xb = x_ref[0]
    wp = w * pack

    denom = None
    for dy in range(_K):
        for dx in range(_K):
            r = jnp.maximum(xb[dy : dy + oh, dx : dx + wp], np.float32(0.0))
            denom = r if denom is None else denom + r
    zero_den = denom == np.float32(0.0)

    y = lax.broadcasted_iota(jnp.int32, (oh, wp), 0)
    i = lax.broadcasted_iota(jnp.int32, (oh, wp), 1)
    s = i // np.int32(w)
    xx = i - s * np.int32(w)
    c = g * np.int32(pack) + s
    base = ((c * oh + y) * ow + xx) * np.int32(9)

    best_score = jnp.full((oh, wp), -jnp.inf, jnp.float32)
    best_val = jnp.zeros((oh, wp), jnp.float32)
    for j in range(9):
        dy, dx = divmod(j, _K)
        p = xb[dy : dy + oh, dx : dx + wp]
        g_noise = _gumbel_from_index(base + np.int32(j))
        prob = jnp.where(zero_den, np.float32(1.0), jnp.maximum(p, 0.0) / denom)
        score = g_noise + jnp.log(prob)
        take = score > best_score
        best_score = jnp.where(take, score, best_score)
        best_val = jnp.where(take, p, best_val)
    o_ref[0] = best_val


@jax.jit
def kernel(x):
    B, C, H, W = x.shape
    oh = H - _K + 1
    ow = W - _K + 1
    N = B * C
    pack = 4 if N % 4 == 0 else 1
    G = N // pack
    Wp = pack * W
    Wpad = -(-(Wp + _K - 1) // 128) * 128
    xp = x.reshape(G, pack, H, W).transpose(0, 2, 1, 3).reshape(G, H, Wp)
    xp = jnp.pad(xp, ((0, 0), (0, 0), (0, Wpad - Wp)))
    out = pl.pallas_call(
        functools.partial(_pool_kernel, oh=oh, ow=ow, w=W, pack=pack),
        grid=(G,),
        in_specs=[pl.BlockSpec((1, H, Wpad), lambda g: (g, 0, 0))],
        out_specs=pl.BlockSpec((1, oh, Wp), lambda g: (g, 0, 0)),
        out_shape=jax.ShapeDtypeStruct((G, oh, Wp), jnp.float32),
        compiler_params=pltpu.CompilerParams(
            dimension_semantics=("parallel",)
        ),
    )(xp)
    out = out.reshape(G, oh, pack, W)[:, :, :, :ow]
    return out.transpose(0, 2, 1, 3).reshape(B, C, oh, ow)
